# baseline (speedup 1.0000x reference)
import functools

import jax
import jax.numpy as jnp
from jax import lax
from jax.experimental import pallas as pl
from jax.experimental.pallas import tpu as pltpu
from jax.experimental.pallas import tpu_sc as plsc

N_ATOMS = 6400000
NUM_GRAPHS = 100000
LANES = 128
FLAT_ROWS = N_ATOMS * 3 // 384
E_ROWS = N_ATOMS // LANES

TC_BLOCK_ROWS = 400
TC_GRID = FLAT_ROWS // TC_BLOCK_ROWS

NUM_CORES = 2
NUM_SUBCORES = 16
NUM_WORKERS = NUM_CORES * NUM_SUBCORES
ROWS_PER_WORKER = 1560
SC_STAGE_ROWS = 104
SC_STAGES = ROWS_PER_WORKER // SC_STAGE_ROWS
TAIL_ROWS = E_ROWS - NUM_WORKERS * ROWS_PER_WORKER
ACC_PER_TILE = 6272
ACC_PAD = NUM_SUBCORES * ACC_PER_TILE


def _forces_energy_body(p_ref, p0_ref, f_ref, e_ref):
    p = p_ref[...]
    p0 = p0_ref[...]
    dr = p - p0
    f_ref[...] = p0 - p
    sq = dr * dr
    c = lax.broadcasted_iota(jnp.int32, (384, LANES), 0)
    a = lax.broadcasted_iota(jnp.int32, (384, LANES), 1)
    t = jnp.where(c // 3 == a, jnp.float32(0.5), jnp.float32(0.0))
    e_ref[...] = lax.dot_general(
        sq, t, (((1,), (0,)), ((), ())), preferred_element_type=jnp.float32)


_forces_energy = pl.pallas_call(
    _forces_energy_body,
    grid=(TC_GRID,),
    in_specs=[
        pl.BlockSpec((TC_BLOCK_ROWS, 384), lambda i: (i, 0)),
        pl.BlockSpec((TC_BLOCK_ROWS, 384), lambda i: (i, 0)),
    ],
    out_specs=[
        pl.BlockSpec((TC_BLOCK_ROWS, 384), lambda i: (i, 0)),
        pl.BlockSpec((TC_BLOCK_ROWS, LANES), lambda i: (i, 0)),
    ],
    out_shape=[
        jax.ShapeDtypeStruct((FLAT_ROWS, 384), jnp.float32),
        jax.ShapeDtypeStruct((E_ROWS, LANES), jnp.float32),
    ],
)


def _segment_sum_body(e_hbm, b_hbm, out_hbm, accum, e_buf, i_buf,
                      te_buf, ti_buf, zbuf):
    c = lax.axis_index("c")
    s = lax.axis_index("s")
    w = s * NUM_CORES + c

    def _zero(j, _):
        zbuf[pl.ds(j * 16, 16)] = jnp.zeros((16,), jnp.float32)
        return 0
    lax.fori_loop(0, ACC_PER_TILE // 16, _zero, 0)
    pltpu.sync_copy(zbuf, accum.at[pl.ds(s * ACC_PER_TILE, ACC_PER_TILE)])
    plsc.subcore_barrier()

    base_row = w * ROWS_PER_WORKER

    def _stage(k, _):
        a0 = (base_row + k * SC_STAGE_ROWS) * LANES
        pltpu.sync_copy(e_hbm.at[pl.ds(a0, SC_STAGE_ROWS * LANES)], e_buf)
        pltpu.sync_copy(b_hbm.at[pl.ds(a0, SC_STAGE_ROWS * LANES)], i_buf)
        pltpu.sync_copy(e_buf, accum.at[i_buf], add=True)
        return 0
    lax.fori_loop(0, SC_STAGES, _stage, 0)

    @pl.when(w == 0)
    def _tail():
        a0 = NUM_WORKERS * ROWS_PER_WORKER * LANES
        pltpu.sync_copy(e_hbm.at[pl.ds(a0, TAIL_ROWS * LANES)],
                        te_buf)
        pltpu.sync_copy(b_hbm.at[pl.ds(a0, TAIL_ROWS * LANES)],
                        ti_buf)
        pltpu.sync_copy(te_buf, accum.at[ti_buf], add=True)

    plsc.subcore_barrier()
    pltpu.sync_copy(accum.at[pl.ds(s * ACC_PER_TILE, ACC_PER_TILE)],
                    out_hbm.at[c, s])


_segment_sum = pl.kernel(
    _segment_sum_body,
    out_type=jax.ShapeDtypeStruct((NUM_CORES, NUM_SUBCORES, ACC_PER_TILE),
                                  jnp.float32),
    mesh=plsc.VectorSubcoreMesh(core_axis_name="c", subcore_axis_name="s"),
    scratch_types=[
        pltpu.VMEM_SHARED((ACC_PAD,), jnp.float32),
        pltpu.VMEM((SC_STAGE_ROWS * LANES,), jnp.float32),
        pltpu.VMEM((SC_STAGE_ROWS * LANES,), jnp.int32),
        pltpu.VMEM((TAIL_ROWS * LANES,), jnp.float32),
        pltpu.VMEM((TAIL_ROWS * LANES,), jnp.int32),
        pltpu.VMEM((ACC_PER_TILE,), jnp.float32),
    ],
    compiler_params=pltpu.CompilerParams(use_tc_tiling_on_sc=True),
)


def _combine_body(p_ref, o_ref):
    o_ref[...] = p_ref[0] + p_ref[1]


_combine = pl.pallas_call(
    _combine_body,
    in_specs=[pl.BlockSpec((NUM_CORES, ACC_PAD // LANES, LANES),
                           lambda: (0, 0, 0))],
    out_specs=pl.BlockSpec((ACC_PAD // LANES, LANES), lambda: (0, 0)),
    out_shape=jax.ShapeDtypeStruct((ACC_PAD // LANES, LANES), jnp.float32),
)


@jax.jit
def kernel(pos, pos0, batch):
    pos2 = pos.reshape(FLAT_ROWS, 384)
    pos02 = pos0.reshape(FLAT_ROWS, 384)
    forces2, e2 = _forces_energy(pos2, pos02)
    partials = _segment_sum(e2.reshape(N_ATOMS), batch)
    combined = _combine(partials.reshape(NUM_CORES, ACC_PAD // LANES, LANES))
    energy = combined.reshape(ACC_PAD)[:NUM_GRAPHS]
    return energy, forces2.reshape(N_ATOMS, 3)


# speedup vs baseline: 48.5021x; 48.5021x over previous
import functools

import jax
import jax.numpy as jnp
from jax import lax
from jax.experimental import pallas as pl
from jax.experimental.pallas import tpu as pltpu
from jax.experimental.pallas import tpu_sc as plsc

N_ATOMS = 6400000
NUM_GRAPHS = 100000
LANES = 128
FLAT_ROWS = N_ATOMS * 3 // 384
E_ROWS = N_ATOMS // LANES

TC_BLOCK_ROWS = 400
TC_GRID = FLAT_ROWS // TC_BLOCK_ROWS

NUM_CORES = 2
NUM_SUBCORES = 16
NUM_WORKERS = NUM_CORES * NUM_SUBCORES
ROWS_PER_WORKER = 1560
SC_STAGE_ROWS = 104
SC_STAGES = ROWS_PER_WORKER // SC_STAGE_ROWS
TAIL_ROWS = E_ROWS - NUM_WORKERS * ROWS_PER_WORKER
ACC_PER_TILE = 6272
ACC_PAD = NUM_SUBCORES * ACC_PER_TILE


TC_BLOCK_ATOMS = TC_BLOCK_ROWS * LANES


def _forces_energy_body(p_ref, p0_ref, f_ref, e_ref):
    p = p_ref[...]
    p0 = p0_ref[...]
    dr = p - p0
    f_ref[...] = p0 - p
    sq = dr * dr
    e_row = (sq[0] + sq[1] + sq[2]) * jnp.float32(0.5)
    e_ref[...] = e_row.reshape(TC_BLOCK_ROWS, LANES)


_forces_energy = pl.pallas_call(
    _forces_energy_body,
    grid=(TC_GRID,),
    in_specs=[
        pl.BlockSpec((3, TC_BLOCK_ATOMS), lambda i: (0, i)),
        pl.BlockSpec((3, TC_BLOCK_ATOMS), lambda i: (0, i)),
    ],
    out_specs=[
        pl.BlockSpec((3, TC_BLOCK_ATOMS), lambda i: (0, i)),
        pl.BlockSpec((TC_BLOCK_ROWS, LANES), lambda i: (i, 0)),
    ],
    out_shape=[
        jax.ShapeDtypeStruct((3, N_ATOMS), jnp.float32),
        jax.ShapeDtypeStruct((E_ROWS, LANES), jnp.float32),
    ],
)


def _segment_sum_body(e_hbm, b_hbm, out_hbm, accum, e_buf, i_buf,
                      te_buf, ti_buf, zbuf):
    c = lax.axis_index("c")
    s = lax.axis_index("s")
    w = s * NUM_CORES + c

    def _zero(j, _):
        zbuf[pl.ds(j * 16, 16)] = jnp.zeros((16,), jnp.float32)
        return 0
    lax.fori_loop(0, ACC_PER_TILE // 16, _zero, 0)
    pltpu.sync_copy(zbuf, accum.at[pl.ds(s * ACC_PER_TILE, ACC_PER_TILE)])
    plsc.subcore_barrier()

    base_row = w * ROWS_PER_WORKER

    def _stage(k, _):
        a0 = (base_row + k * SC_STAGE_ROWS) * LANES
        pltpu.sync_copy(e_hbm.at[pl.ds(a0, SC_STAGE_ROWS * LANES)], e_buf)
        pltpu.sync_copy(b_hbm.at[pl.ds(a0, SC_STAGE_ROWS * LANES)], i_buf)
        pltpu.sync_copy(e_buf, accum.at[i_buf], add=True)
        return 0
    lax.fori_loop(0, SC_STAGES, _stage, 0)

    @pl.when(w == 0)
    def _tail():
        a0 = NUM_WORKERS * ROWS_PER_WORKER * LANES
        pltpu.sync_copy(e_hbm.at[pl.ds(a0, TAIL_ROWS * LANES)],
                        te_buf)
        pltpu.sync_copy(b_hbm.at[pl.ds(a0, TAIL_ROWS * LANES)],
                        ti_buf)
        pltpu.sync_copy(te_buf, accum.at[ti_buf], add=True)

    plsc.subcore_barrier()
    pltpu.sync_copy(accum.at[pl.ds(s * ACC_PER_TILE, ACC_PER_TILE)],
                    out_hbm.at[c, s])


_segment_sum = pl.kernel(
    _segment_sum_body,
    out_type=jax.ShapeDtypeStruct((NUM_CORES, NUM_SUBCORES, ACC_PER_TILE),
                                  jnp.float32),
    mesh=plsc.VectorSubcoreMesh(core_axis_name="c", subcore_axis_name="s"),
    scratch_types=[
        pltpu.VMEM_SHARED((ACC_PAD,), jnp.float32),
        pltpu.VMEM((SC_STAGE_ROWS * LANES,), jnp.float32),
        pltpu.VMEM((SC_STAGE_ROWS * LANES,), jnp.int32),
        pltpu.VMEM((TAIL_ROWS * LANES,), jnp.float32),
        pltpu.VMEM((TAIL_ROWS * LANES,), jnp.int32),
        pltpu.VMEM((ACC_PER_TILE,), jnp.float32),
    ],
    compiler_params=pltpu.CompilerParams(use_tc_tiling_on_sc=True),
)


def _combine_body(p_ref, o_ref):
    o_ref[...] = p_ref[0] + p_ref[1]


_combine = pl.pallas_call(
    _combine_body,
    in_specs=[pl.BlockSpec((NUM_CORES, ACC_PAD // LANES, LANES),
                           lambda: (0, 0, 0))],
    out_specs=pl.BlockSpec((ACC_PAD // LANES, LANES), lambda: (0, 0)),
    out_shape=jax.ShapeDtypeStruct((ACC_PAD // LANES, LANES), jnp.float32),
)


@jax.jit
def kernel(pos, pos0, batch):
    forces_t, e2 = _forces_energy(pos.T, pos0.T)
    partials = _segment_sum(e2.reshape(N_ATOMS), batch)
    combined = _combine(partials.reshape(NUM_CORES, ACC_PAD // LANES, LANES))
    energy = combined.reshape(ACC_PAD)[:NUM_GRAPHS]
    return energy, forces_t.T
